# baseline (speedup 1.0000x reference)
import functools

import jax
import jax.numpy as jnp
from jax import lax
from jax.experimental import pallas as pl
from jax.experimental.pallas import tpu as pltpu
from jax.experimental.pallas import tpu_sc as plsc

CHG = 32
TR = 2048
EPS = 1e-5


def _make_sc_gather(N, V, D):
    info = plsc.get_sparse_core_info()
    NC, NS = info.num_cores, info.num_subcores
    NW = NC * NS
    assert N % (NW * CHG) == 0
    r_per_w = N // NW
    niter = r_per_w // CHG

    mesh = plsc.VectorSubcoreMesh(core_axis_name="c", subcore_axis_name="s")

    @functools.partial(
        pl.kernel,
        mesh=mesh,
        compiler_params=pltpu.CompilerParams(needs_layout_passes=False),
        out_type=jax.ShapeDtypeStruct((N, D), jnp.float32),
        scratch_types=[
            pltpu.VMEM((CHG,), jnp.int32),
            pltpu.VMEM((CHG,), jnp.int32),
            pltpu.VMEM((CHG, D), jnp.float32),
            pltpu.VMEM((CHG, D), jnp.float32),
            pltpu.SemaphoreType.DMA,
            pltpu.SemaphoreType.DMA,
            pltpu.SemaphoreType.DMA,
            pltpu.SemaphoreType.DMA,
        ],
    )
    def sc_gather(ids_h, tok_h, out_h, ix0, ix1, tb0, tb1, g0, g1, o0, o1):
        wid = lax.axis_index("s") * NC + lax.axis_index("c")
        base = wid * r_per_w
        ixs, tbs, gsems, osems = (ix0, ix1), (tb0, tb1), (g0, g1), (o0, o1)

        def stage_gather(t):
            pt = t % 2
            pltpu.sync_copy(ids_h.at[pl.ds(base + t * CHG, CHG)], ixs[pt])
            return pltpu.async_copy(tok_h.at[ixs[pt]], tbs[pt], gsems[pt])

        gh = [None] * niter
        oh = [None] * niter
        gh[0] = stage_gather(0)
        for t in range(niter):
            pt = t % 2
            if t + 1 < niter:
                if t >= 1:
                    oh[t - 1].wait()
                gh[t + 1] = stage_gather(t + 1)
            gh[t].wait()
            oh[t] = pltpu.async_copy(
                tbs[pt], out_h.at[pl.ds(base + t * CHG, CHG)], osems[pt])
        oh[niter - 2].wait()
        oh[niter - 1].wait()

    return sc_gather


def _tc_ln_kernel(tok_ref, pos_ref, segf_ref, seg0_ref, dif_ref,
                  gam_ref, bet_ref, out_ref):
    x = (tok_ref[...] + pos_ref[...] + seg0_ref[...]
         + segf_ref[0, 0][:, None] * dif_ref[...])
    mean = jnp.mean(x, axis=-1, keepdims=True)
    var = jnp.mean(jnp.square(x - mean), axis=-1, keepdims=True)
    y = (x - mean) * lax.rsqrt(var + EPS)
    out_ref[...] = y * gam_ref[...] + bet_ref[...]


def kernel(input_ids, segment_ids, tok_table, pos_table, seg_table,
           ln_gamma, ln_beta):
    B, S = input_ids.shape
    V, D = tok_table.shape
    N = B * S
    ids = input_ids.reshape(N).astype(jnp.int32)
    segf = segment_ids.reshape(N // TR, 1, TR).astype(jnp.float32)
    dif = (seg_table[1] - seg_table[0]).reshape(1, D)

    NH = 2
    H = N // NH
    sc_gather = _make_sc_gather(H, V, D)
    nsb = S // TR

    def tc_ln(gathered_h, segf_h):
        return pl.pallas_call(
            _tc_ln_kernel,
            grid=(H // TR,),
            in_specs=[
                pl.BlockSpec((TR, D), lambda i: (i, 0)),
                pl.BlockSpec((TR, D), lambda i: (i % nsb, 0)),
                pl.BlockSpec((1, 1, TR), lambda i: (i, 0, 0)),
                pl.BlockSpec((1, D), lambda i: (0, 0)),
                pl.BlockSpec((1, D), lambda i: (0, 0)),
                pl.BlockSpec((1, D), lambda i: (0, 0)),
                pl.BlockSpec((1, D), lambda i: (0, 0)),
            ],
            out_specs=pl.BlockSpec((TR, D), lambda i: (i, 0)),
            out_shape=jax.ShapeDtypeStruct((H, D), jnp.float32),
        )(gathered_h, pos_table, segf_h, seg_table[0:1], dif,
          ln_gamma.reshape(1, D), ln_beta.reshape(1, D))

    gathered = [sc_gather(lax.slice_in_dim(ids, h * H, (h + 1) * H),
                          tok_table) for h in range(NH)]
    outs = [tc_ln(gathered[h],
                  lax.slice_in_dim(segf, h * (H // TR), (h + 1) * (H // TR)))
            for h in range(NH)]
    return jnp.concatenate(outs).reshape(B, S, D)


# speedup vs baseline: 1.3030x; 1.3030x over previous
import functools

import jax
import jax.numpy as jnp
from jax import lax
from jax.experimental import pallas as pl
from jax.experimental.pallas import tpu as pltpu
from jax.experimental.pallas import tpu_sc as plsc

CHG = 32
TR = 2048
EPS = 1e-5


def _make_sc_gather(N, V, D):
    info = plsc.get_sparse_core_info()
    NC, NS = info.num_cores, info.num_subcores
    NW = NC * NS
    assert N % (NW * CHG) == 0
    r_per_w = N // NW
    niter = r_per_w // CHG

    mesh = plsc.VectorSubcoreMesh(core_axis_name="c", subcore_axis_name="s")

    @functools.partial(
        pl.kernel,
        mesh=mesh,
        compiler_params=pltpu.CompilerParams(needs_layout_passes=False),
        out_type=jax.ShapeDtypeStruct((N, D), jnp.float32),
        scratch_types=[
            pltpu.VMEM((CHG,), jnp.int32),
            pltpu.VMEM((CHG,), jnp.int32),
            pltpu.VMEM((CHG, D), jnp.float32),
            pltpu.VMEM((CHG, D), jnp.float32),
            pltpu.SemaphoreType.DMA,
            pltpu.SemaphoreType.DMA,
            pltpu.SemaphoreType.DMA,
            pltpu.SemaphoreType.DMA,
        ],
    )
    def sc_gather(ids_h, tok_h, out_h, ix0, ix1, tb0, tb1, g0, g1, o0, o1):
        wid = lax.axis_index("s") * NC + lax.axis_index("c")
        base = wid * r_per_w
        ixs, tbs, gsems, osems = (ix0, ix1), (tb0, tb1), (g0, g1), (o0, o1)

        def stage_gather(t):
            pt = t % 2
            pltpu.sync_copy(ids_h.at[pl.ds(base + t * CHG, CHG)], ixs[pt])
            return pltpu.async_copy(tok_h.at[ixs[pt]], tbs[pt], gsems[pt])

        gh = [None] * niter
        oh = [None] * niter
        gh[0] = stage_gather(0)
        for t in range(niter):
            pt = t % 2
            if t + 1 < niter:
                if t >= 1:
                    oh[t - 1].wait()
                gh[t + 1] = stage_gather(t + 1)
            gh[t].wait()
            oh[t] = pltpu.async_copy(
                tbs[pt], out_h.at[pl.ds(base + t * CHG, CHG)], osems[pt])
        oh[niter - 2].wait()
        oh[niter - 1].wait()

    return sc_gather


def _tc_ln_kernel(tok_ref, pos_ref, segf_ref, seg0_ref, dif_ref,
                  gam_ref, bet_ref, out_ref):
    x = (tok_ref[...] + pos_ref[...] + seg0_ref[...]
         + segf_ref[0, 0][:, None] * dif_ref[...])
    mean = jnp.mean(x, axis=-1, keepdims=True)
    var = jnp.mean(jnp.square(x - mean), axis=-1, keepdims=True)
    y = (x - mean) * lax.rsqrt(var + EPS)
    out_ref[...] = y * gam_ref[...] + bet_ref[...]


def kernel(input_ids, segment_ids, tok_table, pos_table, seg_table,
           ln_gamma, ln_beta):
    B, S = input_ids.shape
    V, D = tok_table.shape
    N = B * S
    ids = input_ids.reshape(N).astype(jnp.int32)
    segf = segment_ids.reshape(N // TR, 1, TR).astype(jnp.float32)

    gathered = _make_sc_gather(N, V, D)(ids, tok_table)

    nsb = S // TR
    out = pl.pallas_call(
        _tc_ln_kernel,
        grid=(N // TR,),
        in_specs=[
            pl.BlockSpec((TR, D), lambda i: (i, 0)),
            pl.BlockSpec((TR, D), lambda i: (i % nsb, 0)),
            pl.BlockSpec((1, 1, TR), lambda i: (i, 0, 0)),
            pl.BlockSpec((1, D), lambda i: (0, 0)),
            pl.BlockSpec((1, D), lambda i: (0, 0)),
            pl.BlockSpec((1, D), lambda i: (0, 0)),
            pl.BlockSpec((1, D), lambda i: (0, 0)),
        ],
        out_specs=pl.BlockSpec((TR, D), lambda i: (i, 0)),
        out_shape=jax.ShapeDtypeStruct((N, D), jnp.float32),
    )(gathered, pos_table, segf,
      seg_table[0:1], (seg_table[1] - seg_table[0]).reshape(1, D),
      ln_gamma.reshape(1, D), ln_beta.reshape(1, D))

    return out.reshape(B, S, D)


# speedup vs baseline: 1.3045x; 1.0011x over previous
import functools

import jax
import jax.numpy as jnp
from jax import lax
from jax.experimental import pallas as pl
from jax.experimental.pallas import tpu as pltpu
from jax.experimental.pallas import tpu_sc as plsc

CHG = 32
TR = 2048
EPS = 1e-5


def _make_sc_gather(N, V, D):
    info = plsc.get_sparse_core_info()
    NC, NS = info.num_cores, info.num_subcores
    NW = NC * NS
    assert N % (NW * CHG) == 0
    r_per_w = N // NW
    niter = r_per_w // CHG

    mesh = plsc.VectorSubcoreMesh(core_axis_name="c", subcore_axis_name="s")

    @functools.partial(
        pl.kernel,
        mesh=mesh,
        compiler_params=pltpu.CompilerParams(needs_layout_passes=False),
        out_type=jax.ShapeDtypeStruct((N, D), jnp.float32),
        scratch_types=[
            pltpu.VMEM((CHG,), jnp.int32),
            pltpu.VMEM((CHG,), jnp.int32),
            pltpu.VMEM((CHG, D), jnp.float32),
            pltpu.VMEM((CHG, D), jnp.float32),
            pltpu.SemaphoreType.DMA,
            pltpu.SemaphoreType.DMA,
            pltpu.SemaphoreType.DMA,
            pltpu.SemaphoreType.DMA,
        ],
    )
    def sc_gather(ids_h, tok_h, out_h, ix0, ix1, tb0, tb1, g0, g1, o0, o1):
        wid = lax.axis_index("s") * NC + lax.axis_index("c")
        base = wid * r_per_w
        ixs, tbs, gsems, osems = (ix0, ix1), (tb0, tb1), (g0, g1), (o0, o1)

        def stage_gather(t):
            pt = t % 2
            pltpu.sync_copy(ids_h.at[pl.ds(base + t * CHG, CHG)], ixs[pt])
            return pltpu.async_copy(tok_h.at[ixs[pt]], tbs[pt], gsems[pt])

        gh = [None] * niter
        oh = [None] * niter
        gh[0] = stage_gather(0)
        for t in range(niter):
            pt = t % 2
            if t + 1 < niter:
                if t >= 1:
                    oh[t - 1].wait()
                gh[t + 1] = stage_gather(t + 1)
            gh[t].wait()
            oh[t] = pltpu.async_copy(
                tbs[pt], out_h.at[pl.ds(base + t * CHG, CHG)], osems[pt])
        oh[niter - 2].wait()
        oh[niter - 1].wait()

    return sc_gather


def _tc_ln_kernel(tok_ref, pos_ref, segf_ref, seg0_ref, dif_ref,
                  gam_ref, bet_ref, out_ref):
    x = (tok_ref[...] + pos_ref[...] + seg0_ref[...]
         + segf_ref[0, 0][:, None] * dif_ref[...])
    mean = jnp.mean(x, axis=-1, keepdims=True)
    msq = jnp.mean(x * x, axis=-1, keepdims=True)
    r = lax.rsqrt(msq - mean * mean + EPS)
    out_ref[...] = (x - mean) * r * gam_ref[...] + bet_ref[...]


def kernel(input_ids, segment_ids, tok_table, pos_table, seg_table,
           ln_gamma, ln_beta):
    B, S = input_ids.shape
    V, D = tok_table.shape
    N = B * S
    ids = input_ids.reshape(N).astype(jnp.int32)
    segf = segment_ids.reshape(N // TR, 1, TR).astype(jnp.float32)

    gathered = _make_sc_gather(N, V, D)(ids, tok_table)

    nsb = S // TR
    out = pl.pallas_call(
        _tc_ln_kernel,
        grid=(N // TR,),
        in_specs=[
            pl.BlockSpec((TR, D), lambda i: (i, 0)),
            pl.BlockSpec((TR, D), lambda i: (i % nsb, 0)),
            pl.BlockSpec((1, 1, TR), lambda i: (i, 0, 0)),
            pl.BlockSpec((1, D), lambda i: (0, 0)),
            pl.BlockSpec((1, D), lambda i: (0, 0)),
            pl.BlockSpec((1, D), lambda i: (0, 0)),
            pl.BlockSpec((1, D), lambda i: (0, 0)),
        ],
        out_specs=pl.BlockSpec((TR, D), lambda i: (i, 0)),
        out_shape=jax.ShapeDtypeStruct((N, D), jnp.float32),
    )(gathered, pos_table, segf,
      seg_table[0:1], (seg_table[1] - seg_table[0]).reshape(1, D),
      ln_gamma.reshape(1, D), ln_beta.reshape(1, D))

    return out.reshape(B, S, D)


# speedup vs baseline: 1.3163x; 1.0090x over previous
import functools

import jax
import jax.numpy as jnp
from jax import lax
from jax.experimental import pallas as pl
from jax.experimental.pallas import tpu as pltpu
from jax.experimental.pallas import tpu_sc as plsc

CHG = 32
TR = 2048
EPS = 1e-5


def _make_sc_gather(N, V, D):
    info = plsc.get_sparse_core_info()
    NC, NS = info.num_cores, info.num_subcores
    NW = NC * NS
    assert N % (NW * CHG) == 0
    r_per_w = N // NW
    niter = r_per_w // CHG

    mesh = plsc.VectorSubcoreMesh(core_axis_name="c", subcore_axis_name="s")

    @functools.partial(
        pl.kernel,
        mesh=mesh,
        compiler_params=pltpu.CompilerParams(needs_layout_passes=False),
        out_type=jax.ShapeDtypeStruct((N, D), jnp.float32),
        scratch_types=[
            pltpu.VMEM((CHG,), jnp.int32),
            pltpu.VMEM((CHG,), jnp.int32),
            pltpu.VMEM((CHG,), jnp.int32),
            pltpu.VMEM((CHG, D), jnp.float32),
            pltpu.VMEM((CHG, D), jnp.float32),
            pltpu.VMEM((CHG, D), jnp.float32),
            pltpu.SemaphoreType.DMA,
            pltpu.SemaphoreType.DMA,
            pltpu.SemaphoreType.DMA,
            pltpu.SemaphoreType.DMA,
            pltpu.SemaphoreType.DMA,
            pltpu.SemaphoreType.DMA,
        ],
    )
    def sc_gather(ids_h, tok_h, out_h, ix0, ix1, ix2, tb0, tb1, tb2,
                  g0, g1, g2, o0, o1, o2):
        wid = lax.axis_index("s") * NC + lax.axis_index("c")
        base = wid * r_per_w
        ixs, tbs = (ix0, ix1, ix2), (tb0, tb1, tb2)
        gsems, osems = (g0, g1, g2), (o0, o1, o2)

        def stage_gather(t):
            pt = t % 3
            pltpu.sync_copy(ids_h.at[pl.ds(base + t * CHG, CHG)], ixs[pt])
            return pltpu.async_copy(tok_h.at[ixs[pt]], tbs[pt], gsems[pt])

        gh = [None] * niter
        oh = [None] * niter
        gh[0] = stage_gather(0)
        gh[1] = stage_gather(1)
        for t in range(niter):
            pt = t % 3
            if t + 2 < niter:
                if t >= 1:
                    oh[t - 1].wait()
                gh[t + 2] = stage_gather(t + 2)
            gh[t].wait()
            oh[t] = pltpu.async_copy(
                tbs[pt], out_h.at[pl.ds(base + t * CHG, CHG)], osems[pt])
        oh[niter - 3].wait()
        oh[niter - 2].wait()
        oh[niter - 1].wait()

    return sc_gather


def _tc_ln_kernel(tok_ref, pos_ref, segf_ref, seg0_ref, dif_ref,
                  gam_ref, bet_ref, out_ref):
    x = (tok_ref[...] + pos_ref[...] + seg0_ref[...]
         + segf_ref[0, 0][:, None] * dif_ref[...])
    mean = jnp.mean(x, axis=-1, keepdims=True)
    msq = jnp.mean(x * x, axis=-1, keepdims=True)
    r = lax.rsqrt(msq - mean * mean + EPS)
    out_ref[...] = (x - mean) * r * gam_ref[...] + bet_ref[...]


def kernel(input_ids, segment_ids, tok_table, pos_table, seg_table,
           ln_gamma, ln_beta):
    B, S = input_ids.shape
    V, D = tok_table.shape
    N = B * S
    ids = input_ids.reshape(N).astype(jnp.int32)
    segf = segment_ids.reshape(N // TR, 1, TR).astype(jnp.float32)

    gathered = _make_sc_gather(N, V, D)(ids, tok_table)

    nsb = S // TR
    out = pl.pallas_call(
        _tc_ln_kernel,
        grid=(N // TR,),
        in_specs=[
            pl.BlockSpec((TR, D), lambda i: (i, 0)),
            pl.BlockSpec((TR, D), lambda i: (i % nsb, 0)),
            pl.BlockSpec((1, 1, TR), lambda i: (i, 0, 0)),
            pl.BlockSpec((1, D), lambda i: (0, 0)),
            pl.BlockSpec((1, D), lambda i: (0, 0)),
            pl.BlockSpec((1, D), lambda i: (0, 0)),
            pl.BlockSpec((1, D), lambda i: (0, 0)),
        ],
        out_specs=pl.BlockSpec((TR, D), lambda i: (i, 0)),
        out_shape=jax.ShapeDtypeStruct((N, D), jnp.float32),
    )(gathered, pos_table, segf,
      seg_table[0:1], (seg_table[1] - seg_table[0]).reshape(1, D),
      ln_gamma.reshape(1, D), ln_beta.reshape(1, D))

    return out.reshape(B, S, D)
